# baseline (speedup 1.0000x reference)
import jax
import jax.numpy as jnp
from jax import lax
from jax.experimental import pallas as pl
from jax.experimental.pallas import tpu as pltpu
from jax.experimental.pallas import tpu_sc as plsc

N = 10000
D = 128
E = 320000
NC, NS, L = 2, 16, 16
NW = NC * NS
CH = 128
NCHUNK = 80
NCA = NCHUNK + 2
ROWS_PER_SUB = 640
NP = NS * ROWS_PER_SUB
DUMMY = N

_mesh = plsc.VectorSubcoreMesh(
    core_axis_name="c", subcore_axis_name="s", num_cores=NC, num_subcores=NS)

_f32 = jnp.float32


def _sc_pass_body(table, ixg, ixs, part_out,
                  acc, ig0, is0, ig1, is1, rb0, rb1,
                  semg0, sems0, semg1, sems1):
    cid = lax.axis_index("c")
    sid = lax.axis_index("s")
    wid = sid * NC + cid
    zeros16 = jnp.zeros((L,), _f32)

    @pl.loop(0, CH)
    def _(i):
        for t in range(D // L):
            rb0[i, pl.ds(t * L, L)] = zeros16

    @pl.loop(0, ROWS_PER_SUB // CH)
    def _(k):
        pltpu.sync_copy(rb0, acc.at[pl.ds(sid * ROWS_PER_SUB + k * CH, CH)])

    pltpu.sync_copy(ixg.at[wid, 0], ig0.at[0])
    pltpu.sync_copy(ixs.at[wid, 0], is0.at[0])
    pltpu.async_copy(table.at[ig0.at[0]], rb0, semg0)
    pltpu.sync_copy(ixg.at[wid, 1], ig1.at[0])
    pltpu.sync_copy(ixs.at[wid, 1], is1.at[0])
    pltpu.async_copy(table.at[ig1.at[0]], rb1, semg1)

    plsc.subcore_barrier()

    @pl.loop(0, NCHUNK // 2)
    def _(k):
        j0 = 2 * k
        pltpu.make_async_copy(table.at[ig0.at[0]], rb0, semg0).wait()
        pltpu.async_copy(rb0, acc.at[is0.at[0]], sems0, add=True)
        pltpu.make_async_copy(table.at[ig1.at[0]], rb1, semg1).wait()
        pltpu.async_copy(rb1, acc.at[is1.at[0]], sems1, add=True)
        pltpu.make_async_copy(rb0, acc.at[is0.at[0]], sems0).wait()
        pltpu.sync_copy(ixg.at[wid, j0 + 2], ig0.at[0])
        pltpu.sync_copy(ixs.at[wid, j0 + 2], is0.at[0])
        pltpu.async_copy(table.at[ig0.at[0]], rb0, semg0)
        pltpu.make_async_copy(rb1, acc.at[is1.at[0]], sems1).wait()
        pltpu.sync_copy(ixg.at[wid, j0 + 3], ig1.at[0])
        pltpu.sync_copy(ixs.at[wid, j0 + 3], is1.at[0])
        pltpu.async_copy(table.at[ig1.at[0]], rb1, semg1)

    pltpu.make_async_copy(table.at[ig0.at[0]], rb0, semg0).wait()
    pltpu.make_async_copy(table.at[ig1.at[0]], rb1, semg1).wait()

    plsc.subcore_barrier()

    pltpu.sync_copy(acc.at[pl.ds(sid * ROWS_PER_SUB, ROWS_PER_SUB)],
                    part_out.at[cid, pl.ds(sid * ROWS_PER_SUB, ROWS_PER_SUB)])


_sc_pass = pl.kernel(
    _sc_pass_body,
    out_type=[jax.ShapeDtypeStruct((NC, NP, D), _f32)],
    mesh=_mesh,
    scratch_types=[
        pltpu.VMEM_SHARED((NP, D), _f32),
        pltpu.VMEM((1, CH), jnp.int32),
        pltpu.VMEM((1, CH), jnp.int32),
        pltpu.VMEM((1, CH), jnp.int32),
        pltpu.VMEM((1, CH), jnp.int32),
        pltpu.VMEM((CH, D), _f32),
        pltpu.VMEM((CH, D), _f32),
        pltpu.SemaphoreType.DMA,
        pltpu.SemaphoreType.DMA,
        pltpu.SemaphoreType.DMA,
        pltpu.SemaphoreType.DMA,
    ],
    compiler_params=pltpu.CompilerParams(needs_layout_passes=False),
)


def _hist_body(ixg, ixs, histg_out, hists_out, slab_g, slab_s, hg, hs):
    cid = lax.axis_index("c")
    sid = lax.axis_index("s")
    wid = sid * NC + cid
    zeros16 = jnp.zeros((L,), _f32)
    ones16 = jnp.ones((L,), _f32)

    @pl.loop(0, NP // L)
    def _(i):
        hg[pl.ds(i * L, L)] = zeros16
        hs[pl.ds(i * L, L)] = zeros16

    pltpu.sync_copy(ixg.at[wid, pl.ds(0, NCHUNK)], slab_g)
    pltpu.sync_copy(ixs.at[wid, pl.ds(0, NCHUNK)], slab_s)

    @pl.loop(0, NCHUNK)
    def _(j):
        for t in range(CH // L):
            plsc.addupdate_scatter(hg, [slab_g[j, pl.ds(t * L, L)]], ones16)
            plsc.addupdate_scatter(hs, [slab_s[j, pl.ds(t * L, L)]], ones16)

    pltpu.sync_copy(hg, histg_out.at[wid])
    pltpu.sync_copy(hs, hists_out.at[wid])


_sc_hist = pl.kernel(
    _hist_body,
    out_type=[jax.ShapeDtypeStruct((NW, NP), _f32),
              jax.ShapeDtypeStruct((NW, NP), _f32)],
    mesh=_mesh,
    scratch_types=[
        pltpu.VMEM((NCHUNK, CH), jnp.int32),
        pltpu.VMEM((NCHUNK, CH), jnp.int32),
        pltpu.VMEM((NP,), _f32),
        pltpu.VMEM((NP,), _f32),
    ],
    compiler_params=pltpu.CompilerParams(needs_layout_passes=False),
)


_BLK = 512


def _mm_body(x_ref, w_ref, o_ref):
    o_ref[...] = jnp.dot(x_ref[...], w_ref[...],
                         preferred_element_type=_f32)


def _matmul(x, w):
    return pl.pallas_call(
        _mm_body,
        grid=(NP // _BLK,),
        in_specs=[pl.BlockSpec((_BLK, D), lambda i: (i, 0)),
                  pl.BlockSpec((D, D), lambda i: (0, 0))],
        out_specs=pl.BlockSpec((_BLK, D), lambda i: (i, 0)),
        out_shape=jax.ShapeDtypeStruct((NP, D), _f32),
    )(x, w)


def _combine_edge_body(p_ref, hb_ref, hd_ref, ef_ref, dinv_ref, binv_ref):
    b = jnp.sum(hb_ref[...], axis=0)
    binv = jnp.where(b > 0, 1.0 / b, 0.0)
    binv_ref[...] = binv
    d = jnp.sum(hd_ref[...], axis=0)
    dinv_ref[...] = jnp.where(d > 0, 1.0 / d, 0.0)
    ef_ref[...] = binv[:, None] * (p_ref[0] + p_ref[1])


def _combine_edge(part, hist_b, hist_d):
    return pl.pallas_call(
        _combine_edge_body,
        grid=(NP // _BLK,),
        in_specs=[pl.BlockSpec((NC, _BLK, D), lambda i: (0, i, 0)),
                  pl.BlockSpec((NW, _BLK), lambda i: (0, i)),
                  pl.BlockSpec((NW, _BLK), lambda i: (0, i))],
        out_specs=[pl.BlockSpec((_BLK, D), lambda i: (i, 0)),
                   pl.BlockSpec((_BLK,), lambda i: (i,)),
                   pl.BlockSpec((_BLK,), lambda i: (i,))],
        out_shape=[jax.ShapeDtypeStruct((NP, D), _f32),
                   jax.ShapeDtypeStruct((NP,), _f32),
                   jax.ShapeDtypeStruct((NP,), _f32)],
    )(part, hist_b, hist_d)


def _combine_edge2_body(p_ref, binv_ref, ef_ref):
    ef_ref[...] = binv_ref[...][:, None] * (p_ref[0] + p_ref[1])


def _combine_edge2(part, binv):
    return pl.pallas_call(
        _combine_edge2_body,
        grid=(NP // _BLK,),
        in_specs=[pl.BlockSpec((NC, _BLK, D), lambda i: (0, i, 0)),
                  pl.BlockSpec((_BLK,), lambda i: (i,))],
        out_specs=pl.BlockSpec((_BLK, D), lambda i: (i, 0)),
        out_shape=jax.ShapeDtypeStruct((NP, D), _f32),
    )(part, binv)


def _combine_node_mm_body(p_ref, dinv_ref, b_ref, w_ref, o_ref):
    i = pl.program_id(0)
    h = dinv_ref[...][:, None] * (p_ref[0] + p_ref[1]) + b_ref[...]
    h = jnp.maximum(h, 0.0)
    rid = i * _BLK + lax.broadcasted_iota(jnp.int32, (_BLK, 1), 0)
    h = jnp.where(rid < N, h, 0.0)
    o_ref[...] = jnp.dot(h, w_ref[...], preferred_element_type=_f32)


def _combine_node_mm(part, dinv, b1, w2):
    return pl.pallas_call(
        _combine_node_mm_body,
        grid=(NP // _BLK,),
        in_specs=[pl.BlockSpec((NC, _BLK, D), lambda i: (0, i, 0)),
                  pl.BlockSpec((_BLK,), lambda i: (i,)),
                  pl.BlockSpec((1, D), lambda i: (0, 0)),
                  pl.BlockSpec((D, D), lambda i: (0, 0))],
        out_specs=pl.BlockSpec((_BLK, D), lambda i: (i, 0)),
        out_shape=jax.ShapeDtypeStruct((NP, D), _f32),
    )(part, dinv, b1.reshape(1, D), w2)


def _final_body(p_ref, dinv_ref, b_ref, o_ref):
    o_ref[...] = (dinv_ref[...][:, None] * (p_ref[0] + p_ref[1])
                  + b_ref[...])


def _final(part, dinv, b2):
    return pl.pallas_call(
        _final_body,
        grid=(NP // _BLK,),
        in_specs=[pl.BlockSpec((NC, _BLK, D), lambda i: (0, i, 0)),
                  pl.BlockSpec((_BLK,), lambda i: (i,)),
                  pl.BlockSpec((1, D), lambda i: (0, 0))],
        out_specs=pl.BlockSpec((_BLK, D), lambda i: (i, 0)),
        out_shape=jax.ShapeDtypeStruct((NP, D), _f32),
    )(part, dinv, b2.reshape(1, D))


def _pad_indices(idx):
    filler = jnp.full((NW * NCHUNK * CH - E,), DUMMY, jnp.int32)
    a = jnp.concatenate([idx.astype(jnp.int32), filler])
    a = a.reshape(NCHUNK, NW, CH).transpose(1, 0, 2)
    tail = jnp.full((NW, NCA - NCHUNK, CH), DUMMY, jnp.int32)
    return jnp.concatenate([a, tail], axis=1)


def kernel(x, hyperedges, hyperedge_attrs, W1, b1, W2, b2):
    del hyperedge_attrs
    ni = _pad_indices(hyperedges[0])
    ei = _pad_indices(hyperedges[1])
    xp = jnp.zeros((NP, D), _f32).at[:N].set(x)

    hist_d, hist_b = _sc_hist(ni, ei)

    x1 = _matmul(xp, W1)
    (epart,) = _sc_pass(x1, ni, ei)
    ef, dinv, binv = _combine_edge(epart, hist_b, hist_d)
    (npart,) = _sc_pass(ef, ei, ni)
    x2 = _combine_node_mm(npart, dinv, b1, W2)

    (epart2,) = _sc_pass(x2, ni, ei)
    ef2 = _combine_edge2(epart2, binv)
    (npart2,) = _sc_pass(ef2, ei, ni)
    return _final(npart2, dinv, b2)[:N]
